# baseline (speedup 1.0000x reference)
import jax
import jax.numpy as jnp
from jax import lax
from jax.experimental import pallas as pl
from jax.experimental.pallas import tpu as pltpu
from jax.experimental.pallas import tpu_sc as plsc

N_NODES = 10000
N_EDGES = 320000
NP = 10240
HALF = NP // 2
NC = 2
NS = 16
CH = 64
NMAXT = 320
SB = 40
NB = 2
ACC_R = 5248
RPT = ACC_R // NS
CAPC = NS * NMAXT * CH


def _agg_body(src_hbm, sidx_hbm, didx_hbm, meta_hbm, out_hbm,
              sidx_v, didx_v, rows_v, meta_v, acc_sh,
              gsem0, gsem1, ssem0, ssem1):
    cid = lax.axis_index("c")
    sid = lax.axis_index("s")
    gsem = (gsem0, gsem1)
    ssem = (ssem0, ssem1)

    pltpu.sync_copy(meta_hbm.at[cid], meta_v)
    t_total = meta_v[0, pl.ds(0, 16)][0]
    ts = t_total // NS
    n_st = (ts + SB - 1) // SB

    def _zrow(r, c):
        for s in range(2):
            for k in range(8):
                rows_v[0, r, s, pl.ds(16 * k, 16)] = jnp.zeros(
                    (16,), jnp.float32)
        return c
    lax.fori_loop(0, CH, _zrow, 0)
    base = sid * RPT
    for i in range(5):
        pltpu.sync_copy(rows_v.at[0], acc_sh.at[pl.ds(base + i * CH, CH)])
    pltpu.sync_copy(rows_v.at[0, pl.ds(0, 8)],
                    acc_sh.at[pl.ds(base + 5 * CH, 8)])
    plsc.subcore_barrier()

    def gather(j, b):
        return pltpu.make_async_copy(
            src_hbm.at[sidx_v.at[j]], rows_v.at[b], gsem[b])

    def scatter(j, b):
        return pltpu.make_async_copy(
            rows_v.at[b], acc_sh.at[didx_v.at[j]], ssem[b])

    def _stage(st, c):
        off = st * SB
        cnt = jnp.minimum(SB, ts - off)
        pltpu.sync_copy(sidx_hbm.at[cid, sid, pl.ds(off, SB)], sidx_v)
        pltpu.sync_copy(didx_hbm.at[cid, sid, pl.ds(off, SB)], didx_v)
        gather(0, 0).start()
        gather(1, 1).start()

        def _t(t, c2):
            for b in range(NB):
                j = t * NB + b

                @pl.when(j >= 1)
                def _():
                    scatter(j - 1, 1 - b).wait()

                @pl.when((j >= 1) & (j + 1 < cnt))
                def _():
                    gather(j + 1, 1 - b).start()

                gather(j, b).wait()
                scatter(j, b).start(add=True)
            return c2
        lax.fori_loop(0, cnt // NB, _t, 0)
        scatter(cnt - 1, 1).wait()
        return c
    lax.fori_loop(0, n_st, _stage, 0)
    plsc.subcore_barrier()

    pltpu.sync_copy(acc_sh.at[pl.ds(base, RPT)],
                    out_hbm.at[cid, pl.ds(base, RPT)])


_AGG = pl.kernel(
    _agg_body,
    mesh=plsc.VectorSubcoreMesh(core_axis_name="c", subcore_axis_name="s"),
    out_type=jax.ShapeDtypeStruct((NC, ACC_R, 2, 128), jnp.float32),
    scratch_types=[
        pltpu.VMEM((SB, CH), jnp.int32),
        pltpu.VMEM((SB, CH), jnp.int32),
        pltpu.VMEM((NB, CH, 2, 128), jnp.float32),
        pltpu.VMEM((8, 128), jnp.int32),
        pltpu.VMEM_SHARED((ACC_R, 2, 128), jnp.float32),
        pltpu.SemaphoreType.DMA,
        pltpu.SemaphoreType.DMA,
        pltpu.SemaphoreType.DMA,
        pltpu.SemaphoreType.DMA,
    ])


BN = 512
GRID = NP // BN


def _wspec(shape):
    return pl.BlockSpec(shape, lambda i: (0,) * len(shape))


def _aspec():
    return pl.BlockSpec((1, BN, 256), lambda i: (i // 10, i % 10, 0))


def _mlp_first(x_pad, agg, W1, b1, W2, b2):
    def body(x_ref, a_ref, w1_ref, b1_ref, w2_ref, b2_ref, o_ref):
        g = x_ref[...] + a_ref[0, :, :128]
        t = jnp.dot(g, w1_ref[...], preferred_element_type=jnp.float32)
        t = jnp.maximum(t + b1_ref[...], 0.0)
        h = jnp.dot(t, w2_ref[...], preferred_element_type=jnp.float32)
        o_ref[...] = jnp.maximum(h + b2_ref[...], 0.0)

    return pl.pallas_call(
        body,
        grid=(GRID,),
        in_specs=[
            pl.BlockSpec((BN, 128), lambda i: (i, 0)),
            _aspec(),
            _wspec((128, 256)), _wspec((1, 256)),
            _wspec((256, 256)), _wspec((1, 256)),
        ],
        out_specs=pl.BlockSpec((BN, 256), lambda i: (i, 0)),
        out_shape=jax.ShapeDtypeStruct((NP, 256), jnp.float32),
    )(x_pad, agg, W1, b1, W2, b2)


def _mlp_mid(h, agg, W3, b3, W4, b4):
    def body(h_ref, a_ref, w3_ref, b3_ref, w4_ref, b4_ref, o_ref):
        g = h_ref[...] + a_ref[0]
        t = jnp.dot(g, w3_ref[...], preferred_element_type=jnp.float32)
        t = jnp.maximum(t + b3_ref[...], 0.0)
        hh = jnp.dot(t, w4_ref[...], preferred_element_type=jnp.float32)
        o_ref[...] = jnp.maximum(hh + b4_ref[...], 0.0)

    return pl.pallas_call(
        body,
        grid=(GRID,),
        in_specs=[
            pl.BlockSpec((BN, 256), lambda i: (i, 0)),
            _aspec(),
            _wspec((256, 256)), _wspec((1, 256)),
            _wspec((256, 256)), _wspec((1, 256)),
        ],
        out_specs=pl.BlockSpec((BN, 256), lambda i: (i, 0)),
        out_shape=jax.ShapeDtypeStruct((NP, 256), jnp.float32),
    )(h, agg, W3, b3, W4, b4)


def _mlp_last(h, agg, W5, b5):
    def body(h_ref, a_ref, w5_ref, b5_ref, o_ref):
        g = h_ref[...] + a_ref[0]
        o_ref[...] = jnp.dot(
            g, w5_ref[...], preferred_element_type=jnp.float32) + b5_ref[...]

    return pl.pallas_call(
        body,
        grid=(GRID,),
        in_specs=[
            pl.BlockSpec((BN, 256), lambda i: (i, 0)),
            _aspec(),
            _wspec((256, 128)), _wspec((1, 128)),
        ],
        out_specs=pl.BlockSpec((BN, 128), lambda i: (i, 0)),
        out_shape=jax.ShapeDtypeStruct((NP, 128), jnp.float32),
    )(h, agg, W5, b5)


def kernel(x, edge_index, W1, b1, W2, b2, W3, b3, W4, b4, W5, b5):
    src = edge_index[0].astype(jnp.int32)
    dst = edge_index[1].astype(jnp.int32)

    m = (dst >= HALF).astype(jnp.int32)
    c1 = jnp.cumsum(m)
    rank = jnp.where(m == 1, c1 - 1,
                     jnp.arange(N_EDGES, dtype=jnp.int32) - c1)
    k = rank // CH
    lane = rank % CH
    pos = m * CAPC + (k % NS) * (NMAXT * CH) + (k // NS) * CH + lane

    fill = (jnp.arange(NC * CAPC, dtype=jnp.int32) * 37) % 4096
    sidx = fill.at[pos].set(src, unique_indices=True)
    didx = jnp.full((NC * CAPC,), HALF, jnp.int32).at[pos].set(
        dst - m * HALF, unique_indices=True)
    sidx = sidx.reshape(NC, NS, NMAXT, CH)
    didx = didx.reshape(NC, NS, NMAXT, CH)

    n1 = c1[-1]
    n0 = N_EDGES - n1
    def _tpad(n):
        return (((n + CH - 1) // CH) + 2 * NS - 1) // (2 * NS) * (2 * NS)
    lane0 = (jnp.arange(128) == 0).astype(jnp.int32)
    meta = (jnp.stack([_tpad(n0), _tpad(n1)]).reshape(NC, 1, 1).astype(
        jnp.int32) * lane0.reshape(1, 1, 128)) * jnp.ones(
        (NC, 8, 128), jnp.int32)

    x_pad = jnp.pad(x, ((0, NP - N_NODES), (0, 0)))
    x2 = jnp.concatenate([x_pad, x_pad], axis=1).reshape(NP, 2, 128)
    b1r, b2r, b3r, b4r, b5r = (
        b.reshape(1, -1) for b in (b1, b2, b3, b4, b5))

    agg1 = _AGG(x2, sidx, didx, meta).reshape(NC, ACC_R, 256)
    h1 = _mlp_first(x_pad, agg1, W1, b1r, W2, b2r)
    agg2 = _AGG(h1.reshape(NP, 2, 128), sidx, didx, meta).reshape(
        NC, ACC_R, 256)
    h2 = _mlp_mid(h1, agg2, W3, b3r, W4, b4r)
    agg3 = _AGG(h2.reshape(NP, 2, 128), sidx, didx, meta).reshape(
        NC, ACC_R, 256)
    out = _mlp_last(h2, agg3, W5, b5r)
    return out[:N_NODES]


# speedup vs baseline: 1.7962x; 1.7962x over previous
import jax
import jax.numpy as jnp
from jax import lax
from jax.experimental import pallas as pl
from jax.experimental.pallas import tpu as pltpu
from jax.experimental.pallas import tpu_sc as plsc

N_NODES = 10000
N_EDGES = 320000
NP = 10240
NC = 2
NS = 16
ROWS_PER_TILE = NP // NS
CH = 128
N1 = 80
N2 = 160
ZR = 160


def _make_sc_agg(n_chunks):
    mesh = plsc.VectorSubcoreMesh(core_axis_name="c", subcore_axis_name="s")

    SH = 40
    n_stages = n_chunks // SH
    NB = 2

    def body(src_hbm, sidx_hbm, didx_hbm, out_hbm,
             sidx_v, didx_v, rows_v, acc_sh, gsem0, gsem1, ssem0, ssem1):
        cid = lax.axis_index("c")
        sid = lax.axis_index("s")
        gsem = (gsem0, gsem1)
        ssem = (ssem0, ssem1)

        def gather(j, b):
            return pltpu.make_async_copy(
                src_hbm.at[sidx_v.at[j]], rows_v.at[b], gsem[b])

        def scatter(j, b):
            return pltpu.make_async_copy(
                rows_v.at[b], acc_sh.at[didx_v.at[j]], ssem[b])

        def _zrow(r, c):
            for k in range(8):
                rows_v[0, r, pl.ds(16 * k, 16)] = jnp.zeros((16,),
                                                            jnp.float32)
            return c
        lax.fori_loop(0, CH, _zrow, 0)
        base = sid * ROWS_PER_TILE

        def _zcp(i, c):
            pltpu.sync_copy(rows_v.at[0], acc_sh.at[pl.ds(base + i * CH, CH)])
            return c
        lax.fori_loop(0, ROWS_PER_TILE // CH, _zcp, 0)
        plsc.subcore_barrier()

        for stage in range(n_stages):
            pltpu.sync_copy(sidx_hbm.at[cid, sid, pl.ds(stage * SH, SH)],
                            sidx_v)
            pltpu.sync_copy(didx_hbm.at[cid, sid, pl.ds(stage * SH, SH)],
                            didx_v)
            for b in range(NB):
                gather(b, b).start()

            def _t(t, c):
                for b in range(NB):
                    j = t * NB + b
                    pb = (b - 1) % NB

                    @pl.when(j >= 1)
                    def _():
                        scatter(j - 1, pb).wait()

                    @pl.when((j >= 1) & (j - 1 + NB < SH))
                    def _():
                        gather(j - 1 + NB, pb).start()

                    gather(j, b).wait()
                    scatter(j, b).start(add=True)
                return c
            lax.fori_loop(0, SH // NB, _t, 0)
            scatter(SH - 1, (SH - 1) % NB).wait()
        plsc.subcore_barrier()

        pltpu.sync_copy(acc_sh.at[pl.ds(base, ROWS_PER_TILE)],
                        out_hbm.at[cid, pl.ds(base, ROWS_PER_TILE)])

    return pl.kernel(
        body, mesh=mesh,
        out_type=jax.ShapeDtypeStruct((NC, NP, 128), jnp.float32),
        scratch_types=[
            pltpu.VMEM((SH, CH), jnp.int32),
            pltpu.VMEM((SH, CH), jnp.int32),
            pltpu.VMEM((NB, CH, 128), jnp.float32),
            pltpu.VMEM_SHARED((NP, 128), jnp.float32),
            pltpu.SemaphoreType.DMA,
            pltpu.SemaphoreType.DMA,
            pltpu.SemaphoreType.DMA,
            pltpu.SemaphoreType.DMA,
        ])


_sc_agg_cache = {}


def _sc_agg(n_chunks, src_arr, sidx, didx):
    if n_chunks not in _sc_agg_cache:
        _sc_agg_cache[n_chunks] = _make_sc_agg(n_chunks)
    return _sc_agg_cache[n_chunks](src_arr, sidx, didx)


BN = 512
GRID = NP // BN


def _wspec(shape):
    return pl.BlockSpec(shape, lambda i: (0,) * len(shape))


def _mlp_first(x_pad, acc, W1, b1, W2, b2):
    def body(x_ref, a_ref, w1_ref, b1_ref, w2_ref, b2_ref, o_ref):
        g = x_ref[...] + a_ref[0] + a_ref[1]
        t = jnp.dot(g, w1_ref[...], preferred_element_type=jnp.float32)
        t = jnp.maximum(t + b1_ref[...], 0.0)
        h = jnp.dot(t, w2_ref[...], preferred_element_type=jnp.float32)
        h = jnp.maximum(h + b2_ref[...], 0.0)
        o_ref[0] = h[:, :128]
        o_ref[1] = h[:, 128:]

    return pl.pallas_call(
        body,
        grid=(GRID,),
        in_specs=[
            pl.BlockSpec((BN, 128), lambda i: (i, 0)),
            pl.BlockSpec((2, BN, 128), lambda i: (0, i, 0)),
            _wspec((128, 256)), _wspec((1, 256)),
            _wspec((256, 256)), _wspec((1, 256)),
        ],
        out_specs=pl.BlockSpec((2, BN, 128), lambda i: (0, i, 0)),
        out_shape=jax.ShapeDtypeStruct((2, NP, 128), jnp.float32),
    )(x_pad, acc, W1, b1, W2, b2)


def _mlp_mid(h, acc, W3, b3, W4, b4):
    def body(h_ref, a_ref, w3_ref, b3_ref, w4_ref, b4_ref, o_ref):
        g = jnp.concatenate([h_ref[0] + a_ref[0], h_ref[1] + a_ref[1]], axis=1)
        t = jnp.dot(g, w3_ref[...], preferred_element_type=jnp.float32)
        t = jnp.maximum(t + b3_ref[...], 0.0)
        hh = jnp.dot(t, w4_ref[...], preferred_element_type=jnp.float32)
        hh = jnp.maximum(hh + b4_ref[...], 0.0)
        o_ref[0] = hh[:, :128]
        o_ref[1] = hh[:, 128:]

    return pl.pallas_call(
        body,
        grid=(GRID,),
        in_specs=[
            pl.BlockSpec((2, BN, 128), lambda i: (0, i, 0)),
            pl.BlockSpec((2, BN, 128), lambda i: (0, i, 0)),
            _wspec((256, 256)), _wspec((1, 256)),
            _wspec((256, 256)), _wspec((1, 256)),
        ],
        out_specs=pl.BlockSpec((2, BN, 128), lambda i: (0, i, 0)),
        out_shape=jax.ShapeDtypeStruct((2, NP, 128), jnp.float32),
    )(h, acc, W3, b3, W4, b4)


def _mlp_last(h, acc, W5, b5):
    def body(h_ref, a_ref, w5_ref, b5_ref, o_ref):
        g = jnp.concatenate([h_ref[0] + a_ref[0], h_ref[1] + a_ref[1]], axis=1)
        o_ref[...] = jnp.dot(
            g, w5_ref[...], preferred_element_type=jnp.float32) + b5_ref[...]

    return pl.pallas_call(
        body,
        grid=(GRID,),
        in_specs=[
            pl.BlockSpec((2, BN, 128), lambda i: (0, i, 0)),
            pl.BlockSpec((2, BN, 128), lambda i: (0, i, 0)),
            _wspec((256, 128)), _wspec((1, 128)),
        ],
        out_specs=pl.BlockSpec((BN, 128), lambda i: (i, 0)),
        out_shape=jax.ShapeDtypeStruct((NP, 128), jnp.float32),
    )(h, acc, W5, b5)


def kernel(x, edge_index, W1, b1, W2, b2, W3, b3, W4, b4, W5, b5):
    src = edge_index[0].astype(jnp.int32)
    dst = edge_index[1].astype(jnp.int32)

    e1 = NC * NS * N1 * CH
    sidx1 = jnp.concatenate(
        [src, jnp.zeros((e1 - N_EDGES,), jnp.int32)]).reshape(NC, NS, N1, CH)
    didx1 = jnp.concatenate(
        [dst, jnp.full((e1 - N_EDGES,), N_NODES, jnp.int32)]
    ).reshape(NC, NS, N1, CH)

    e2 = NS * N2 * CH
    s2 = jnp.concatenate(
        [src, jnp.zeros((e2 - N_EDGES,), jnp.int32)]).reshape(1, NS, N2, CH)
    sidx2 = s2 + (jnp.arange(NC, dtype=jnp.int32) * NP).reshape(NC, 1, 1, 1)
    d2 = jnp.concatenate(
        [dst, jnp.full((e2 - N_EDGES,), N_NODES, jnp.int32)]
    ).reshape(1, NS, N2, CH)
    didx2 = jnp.broadcast_to(d2, (NC, NS, N2, CH))

    x_pad = jnp.pad(x, ((0, NP - N_NODES), (0, 0)))
    b1r, b2r, b3r, b4r, b5r = (
        b.reshape(1, -1) for b in (b1, b2, b3, b4, b5))

    acc1 = _sc_agg(N1, x, sidx1, didx1)
    h1 = _mlp_first(x_pad, acc1, W1, b1r, W2, b2r)
    acc2 = _sc_agg(N2, h1.reshape(2 * NP, 128), sidx2, didx2)
    h2 = _mlp_mid(h1, acc2, W3, b3r, W4, b4r)
    acc3 = _sc_agg(N2, h2.reshape(2 * NP, 128), sidx2, didx2)
    out = _mlp_last(h2, acc3, W5, b5r)
    return out[:N_NODES]


# speedup vs baseline: 1.9236x; 1.0709x over previous
import jax
import jax.numpy as jnp
from jax import lax
from jax.experimental import pallas as pl
from jax.experimental.pallas import tpu as pltpu
from jax.experimental.pallas import tpu_sc as plsc

N_NODES = 10000
N_EDGES = 320000
NP = 10240
NC = 2
NS = 16
ROWS_PER_TILE = NP // NS
CH = 128
N1 = 80
N2 = 160
ZR = 160


def _make_sc_agg(n_chunks):
    mesh = plsc.VectorSubcoreMesh(core_axis_name="c", subcore_axis_name="s")

    SH = 40
    n_stages = n_chunks // SH
    NB = 2

    def body(src_hbm, sidx_hbm, didx_hbm, out_hbm,
             sidx_v, didx_v, rows_v, acc_sh, gsem0, gsem1, ssem0, ssem1):
        cid = lax.axis_index("c")
        sid = lax.axis_index("s")
        gsem = (gsem0, gsem1)
        ssem = (ssem0, ssem1)

        def gather(j, b):
            return pltpu.make_async_copy(
                src_hbm.at[sidx_v.at[j]], rows_v.at[b], gsem[b])

        def scatter(j, b):
            return pltpu.make_async_copy(
                rows_v.at[b], acc_sh.at[didx_v.at[j]], ssem[b])

        def _zrow(r, c):
            for k in range(8):
                rows_v[0, r, pl.ds(16 * k, 16)] = jnp.zeros((16,),
                                                            jnp.float32)
            return c
        lax.fori_loop(0, CH, _zrow, 0)
        base = sid * ROWS_PER_TILE

        def _zcp(i, c):
            pltpu.sync_copy(rows_v.at[0], acc_sh.at[pl.ds(base + i * CH, CH)])
            return c
        lax.fori_loop(0, ROWS_PER_TILE // CH, _zcp, 0)
        plsc.subcore_barrier()

        for stage in range(n_stages):
            pltpu.sync_copy(sidx_hbm.at[cid, sid, pl.ds(stage * SH, SH)],
                            sidx_v)
            pltpu.sync_copy(didx_hbm.at[cid, sid, pl.ds(stage * SH, SH)],
                            didx_v)
            for b in range(NB):
                gather(b, b).start()

            def _t(t, c):
                for b in range(NB):
                    j = t * NB + b
                    pb = (b - 1) % NB

                    @pl.when(j >= 1)
                    def _():
                        scatter(j - 1, pb).wait()

                    @pl.when((j >= 1) & (j - 1 + NB < SH))
                    def _():
                        gather(j - 1 + NB, pb).start()

                    gather(j, b).wait()
                    scatter(j, b).start(add=True)
                return c
            lax.fori_loop(0, SH // NB, _t, 0)
            scatter(SH - 1, (SH - 1) % NB).wait()
        plsc.subcore_barrier()

        pltpu.sync_copy(acc_sh.at[pl.ds(base, ROWS_PER_TILE)],
                        out_hbm.at[cid, pl.ds(base, ROWS_PER_TILE)])

    return pl.kernel(
        body, mesh=mesh,
        out_type=jax.ShapeDtypeStruct((NC, NP, 128), jnp.float32),
        scratch_types=[
            pltpu.VMEM((SH, CH), jnp.int32),
            pltpu.VMEM((SH, CH), jnp.int32),
            pltpu.VMEM((NB, CH, 128), jnp.float32),
            pltpu.VMEM_SHARED((NP, 128), jnp.float32),
            pltpu.SemaphoreType.DMA,
            pltpu.SemaphoreType.DMA,
            pltpu.SemaphoreType.DMA,
            pltpu.SemaphoreType.DMA,
        ])


_sc_agg_cache = {}


def _sc_agg(n_chunks, src_arr, sidx, didx):
    if n_chunks not in _sc_agg_cache:
        _sc_agg_cache[n_chunks] = _make_sc_agg(n_chunks)
    return _sc_agg_cache[n_chunks](src_arr, sidx, didx)


BN = 512
GRID = NP // BN


def _wspec(shape):
    return pl.BlockSpec(shape, lambda i: (0,) * len(shape))


def _mlp_first(x_pad, acc, W1, b1, W2, b2):
    def body(x_ref, a_ref, w1_ref, b1_ref, w2_ref, b2_ref, o_ref):
        g = x_ref[...] + a_ref[0] + a_ref[1]
        t = jnp.dot(g, w1_ref[...], preferred_element_type=jnp.float32)
        t = jnp.maximum(t + b1_ref[...], 0.0)
        h = jnp.dot(t, w2_ref[...], preferred_element_type=jnp.float32)
        h = jnp.maximum(h + b2_ref[...], 0.0)
        o_ref[0] = h[:, :128]
        o_ref[1] = h[:, 128:]

    return pl.pallas_call(
        body,
        grid=(GRID,),
        in_specs=[
            pl.BlockSpec((BN, 128), lambda i: (i, 0)),
            pl.BlockSpec((2, BN, 128), lambda i: (0, i, 0)),
            _wspec((128, 256)), _wspec((1, 256)),
            _wspec((256, 256)), _wspec((1, 256)),
        ],
        out_specs=pl.BlockSpec((2, BN, 128), lambda i: (0, i, 0)),
        out_shape=jax.ShapeDtypeStruct((2, NP, 128), jnp.float32),
    )(x_pad, acc, W1, b1, W2, b2)


def _mlp_mid(h, acc, W3, b3, W4, b4, W5):
    def body(h_ref, a_ref, w3_ref, b3_ref, w4_ref, b4_ref, w5_ref, o_ref):
        g = jnp.concatenate([h_ref[0] + a_ref[0], h_ref[1] + a_ref[1]], axis=1)
        t = jnp.dot(g, w3_ref[...], preferred_element_type=jnp.float32)
        t = jnp.maximum(t + b3_ref[...], 0.0)
        hh = jnp.dot(t, w4_ref[...], preferred_element_type=jnp.float32)
        hh = jnp.maximum(hh + b4_ref[...], 0.0)
        o_ref[...] = jnp.dot(hh, w5_ref[...],
                             preferred_element_type=jnp.float32)

    return pl.pallas_call(
        body,
        grid=(GRID,),
        in_specs=[
            pl.BlockSpec((2, BN, 128), lambda i: (0, i, 0)),
            pl.BlockSpec((2, BN, 128), lambda i: (0, i, 0)),
            _wspec((256, 256)), _wspec((1, 256)),
            _wspec((256, 256)), _wspec((1, 256)),
            _wspec((256, 128)),
        ],
        out_specs=pl.BlockSpec((BN, 128), lambda i: (i, 0)),
        out_shape=jax.ShapeDtypeStruct((NP, 128), jnp.float32),
    )(h, acc, W3, b3, W4, b4, W5)


def _mlp_last(p, acc, b5):
    def body(p_ref, a_ref, b5_ref, o_ref):
        o_ref[...] = p_ref[...] + a_ref[0] + a_ref[1] + b5_ref[...]

    return pl.pallas_call(
        body,
        grid=(GRID,),
        in_specs=[
            pl.BlockSpec((BN, 128), lambda i: (i, 0)),
            pl.BlockSpec((2, BN, 128), lambda i: (0, i, 0)),
            _wspec((1, 128)),
        ],
        out_specs=pl.BlockSpec((BN, 128), lambda i: (i, 0)),
        out_shape=jax.ShapeDtypeStruct((NP, 128), jnp.float32),
    )(p, acc, b5)


def kernel(x, edge_index, W1, b1, W2, b2, W3, b3, W4, b4, W5, b5):
    src = edge_index[0].astype(jnp.int32)
    dst = edge_index[1].astype(jnp.int32)

    e1 = NC * NS * N1 * CH
    sidx1 = jnp.concatenate(
        [src, jnp.zeros((e1 - N_EDGES,), jnp.int32)]).reshape(NC, NS, N1, CH)
    didx1 = jnp.concatenate(
        [dst, jnp.full((e1 - N_EDGES,), N_NODES, jnp.int32)]
    ).reshape(NC, NS, N1, CH)

    e2 = NS * N2 * CH
    s2 = jnp.concatenate(
        [src, jnp.zeros((e2 - N_EDGES,), jnp.int32)]).reshape(1, NS, N2, CH)
    sidx2 = s2 + (jnp.arange(NC, dtype=jnp.int32) * NP).reshape(NC, 1, 1, 1)
    d2 = jnp.concatenate(
        [dst, jnp.full((e2 - N_EDGES,), N_NODES, jnp.int32)]
    ).reshape(1, NS, N2, CH)
    didx2 = jnp.broadcast_to(d2, (NC, NS, N2, CH))

    x_pad = jnp.pad(x, ((0, NP - N_NODES), (0, 0)))
    b1r, b2r, b3r, b4r, b5r = (
        b.reshape(1, -1) for b in (b1, b2, b3, b4, b5))

    acc1 = _sc_agg(N1, x, sidx1, didx1)
    h1 = _mlp_first(x_pad, acc1, W1, b1r, W2, b2r)
    acc2 = _sc_agg(N2, h1.reshape(2 * NP, 128), sidx2, didx2)
    p2 = _mlp_mid(h1, acc2, W3, b3r, W4, b4r, W5)
    acc3 = _sc_agg(N1, p2, sidx1, didx1)
    out = _mlp_last(p2, acc3, b5r)
    return out[:N_NODES]
